# baseline (speedup 1.0000x reference)
import functools
import jax
import jax.numpy as jnp
from jax import lax
from jax.experimental import pallas as pl
from jax.experimental.pallas import tpu as pltpu
from jax.experimental.pallas import tpu_sc as plsc

_NUM_CLASSES = 1000000
_DIM = 64
_BATCH = 16384

_info = plsc.get_sparse_core_info()
_NC, _NS = _info.num_cores, _info.num_subcores
_NW = _NC * _NS
_B_PER_W = _BATCH // _NW
_CHUNK = 128
_N_CHUNKS = _B_PER_W // _CHUNK
_PASS_CHUNKS = 2

_TCOLS = 512
_TGRID = -(-_NUM_CLASSES // _TCOLS)

_mesh = plsc.VectorSubcoreMesh(core_axis_name="c", subcore_axis_name="s")


def _tc_transpose_kernel(x_ref, o_ref):
    xt = x_ref[...].T
    o_ref[:, 0:_DIM] = xt[0:_TCOLS // 2, :]
    o_ref[:, _DIM:2 * _DIM] = xt[_TCOLS // 2:_TCOLS, :]


def _tc_transpose(table_t):
    return pl.pallas_call(
        _tc_transpose_kernel,
        grid=(_TGRID,),
        in_specs=[pl.BlockSpec((_DIM, _TCOLS), lambda j: (0, j))],
        out_specs=pl.BlockSpec((_TCOLS // 2, 2 * _DIM), lambda j: (j, 0)),
        out_shape=jax.ShapeDtypeStruct(
            (_TGRID * _TCOLS // 2, 2 * _DIM), jnp.float32
        ),
    )(table_t)


@functools.partial(
    pl.kernel,
    mesh=_mesh,
    out_type=jax.ShapeDtypeStruct((_BATCH, _DIM), jnp.float32),
    scratch_types=[
        pltpu.VMEM((_B_PER_W,), jnp.int32),
        pltpu.VMEM((_B_PER_W,), jnp.int32),
        pltpu.VMEM((_PASS_CHUNKS * _CHUNK, 2 * _DIM), jnp.float32),
        pltpu.VMEM((_B_PER_W, _DIM), jnp.float32),
        pltpu.SemaphoreType.DMA,
    ],
)
def _embed(y_hbm, table2_hbm, out_hbm, idx_v, q_v, gbuf, rows_v, sem):
    wid = lax.axis_index("s") * _NC + lax.axis_index("c")
    base = wid * _B_PER_W
    pltpu.sync_copy(y_hbm.at[pl.ds(base, _B_PER_W)], idx_v)

    def q_body(g, _):
        sl = pl.ds(g * 16, 16)
        vec = idx_v[sl]
        blk = jax.lax.shift_right_logical(vec, 9)
        q_v[sl] = blk * 256 + jax.lax.bitwise_and(vec, 255)
        return _

    lax.fori_loop(0, _B_PER_W // 16, q_body, None)

    for p in range(_N_CHUNKS // _PASS_CHUNKS):
        pbase = p * _PASS_CHUNKS * _CHUNK
        copies = []
        for j in range(_PASS_CHUNKS):
            copies.append(
                pltpu.async_copy(
                    table2_hbm.at[q_v.at[pl.ds(pbase + j * _CHUNK, _CHUNK)]],
                    gbuf.at[pl.ds(j * _CHUNK, _CHUNK)],
                    sem,
                )
            )
        for c in copies:
            c.wait()

        def sel_body(g, _):
            vec = idx_v[pl.ds(pbase + g * 16, 16)]
            for k in range(16):
                j = g * 16 + k
                h = jax.lax.bitwise_and(
                    jax.lax.shift_right_logical(vec[k], 8), 1
                ) * _DIM
                for q in range(4):
                    rows_v[pbase + j, pl.ds(q * 16, 16)] = gbuf[
                        j, pl.ds(h + q * 16, 16)
                    ]
            return _

        lax.fori_loop(0, _PASS_CHUNKS * _CHUNK // 16, sel_body, None)

    pltpu.sync_copy(rows_v, out_hbm.at[pl.ds(base, _B_PER_W)])


def kernel(y, emb_weight):
    assert y.shape == (_BATCH,) and emb_weight.shape == (_NUM_CLASSES, _DIM)
    table2 = _tc_transpose(emb_weight.T)
    return _embed(y.astype(jnp.int32), table2)


# speedup vs baseline: 1.6401x; 1.6401x over previous
import functools
import jax
import jax.numpy as jnp
from jax import lax
from jax.experimental import pallas as pl
from jax.experimental.pallas import tpu as pltpu
from jax.experimental.pallas import tpu_sc as plsc

_NUM_CLASSES = 1000000
_DIM = 64
_BATCH = 16384

_info = plsc.get_sparse_core_info()
_NC, _NS = _info.num_cores, _info.num_subcores
_NW = _NC * _NS
_B_PER_W = _BATCH // _NW
_CHUNK = 128
_N_CHUNKS = _B_PER_W // _CHUNK
_PASS_CHUNKS = 2

_TCOLS = 1024
_THALF = _TCOLS // 2
_TGRID = -(-_NUM_CLASSES // _TCOLS)

_mesh = plsc.VectorSubcoreMesh(core_axis_name="c", subcore_axis_name="s")


def _tc_transpose_kernel(x_ref, o_ref):
    eye = jnp.eye(_DIM, dtype=jnp.float32)
    xt = jax.lax.dot_general(
        x_ref[...], eye,
        dimension_numbers=(((0,), (0,)), ((), ())),
        preferred_element_type=jnp.float32,
    )
    o_ref[:, 0:_DIM] = xt[0:_THALF, :]
    o_ref[:, _DIM:2 * _DIM] = xt[_THALF:_TCOLS, :]


def _tc_transpose(table_t):
    return pl.pallas_call(
        _tc_transpose_kernel,
        grid=(_TGRID,),
        in_specs=[pl.BlockSpec((_DIM, _TCOLS), lambda j: (0, j))],
        out_specs=pl.BlockSpec((_TCOLS // 2, 2 * _DIM), lambda j: (j, 0)),
        out_shape=jax.ShapeDtypeStruct(
            (_TGRID * _TCOLS // 2, 2 * _DIM), jnp.float32
        ),
    )(table_t)


@functools.partial(
    pl.kernel,
    mesh=_mesh,
    out_type=jax.ShapeDtypeStruct((_BATCH, _DIM), jnp.float32),
    scratch_types=[
        pltpu.VMEM((_B_PER_W,), jnp.int32),
        pltpu.VMEM((_B_PER_W,), jnp.int32),
        pltpu.VMEM((_PASS_CHUNKS * _CHUNK, 2 * _DIM), jnp.float32),
        pltpu.VMEM((_B_PER_W, _DIM), jnp.float32),
        pltpu.SemaphoreType.DMA,
    ],
)
def _embed(y_hbm, table2_hbm, out_hbm, idx_v, q_v, gbuf, rows_v, sem):
    wid = lax.axis_index("s") * _NC + lax.axis_index("c")
    base = wid * _B_PER_W
    pltpu.sync_copy(y_hbm.at[pl.ds(base, _B_PER_W)], idx_v)

    def q_body(g, _):
        sl = pl.ds(g * 16, 16)
        vec = idx_v[sl]
        blk = jax.lax.shift_right_logical(vec, 10)
        q_v[sl] = blk * _THALF + jax.lax.bitwise_and(vec, _THALF - 1)
        return _

    lax.fori_loop(0, _B_PER_W // 16, q_body, None)

    for p in range(_N_CHUNKS // _PASS_CHUNKS):
        pbase = p * _PASS_CHUNKS * _CHUNK
        copies = []
        for j in range(_PASS_CHUNKS):
            copies.append(
                pltpu.async_copy(
                    table2_hbm.at[q_v.at[pl.ds(pbase + j * _CHUNK, _CHUNK)]],
                    gbuf.at[pl.ds(j * _CHUNK, _CHUNK)],
                    sem,
                )
            )
        for c in copies:
            c.wait()

        def sel_body(g, _):
            vec = idx_v[pl.ds(pbase + g * 16, 16)]
            for k in range(16):
                j = g * 16 + k
                h = jax.lax.bitwise_and(
                    jax.lax.shift_right_logical(vec[k], 9), 1
                ) * _DIM
                for q in range(4):
                    rows_v[pbase + j, pl.ds(q * 16, 16)] = gbuf[
                        j, pl.ds(h + q * 16, 16)
                    ]
            return _

        lax.fori_loop(0, _PASS_CHUNKS * _CHUNK // 16, sel_body, None)

    pltpu.sync_copy(rows_v, out_hbm.at[pl.ds(base, _B_PER_W)])


def kernel(y, emb_weight):
    assert y.shape == (_BATCH,) and emb_weight.shape == (_NUM_CLASSES, _DIM)
    table2 = _tc_transpose(emb_weight.T)
    return _embed(y.astype(jnp.int32), table2)


# speedup vs baseline: 1.8967x; 1.1565x over previous
import functools
import jax
import jax.numpy as jnp
from jax import lax
from jax.experimental import pallas as pl
from jax.experimental.pallas import tpu as pltpu
from jax.experimental.pallas import tpu_sc as plsc

_NUM_CLASSES = 1000000
_DIM = 64
_BATCH = 16384

_info = plsc.get_sparse_core_info()
_NC, _NS = _info.num_cores, _info.num_subcores
_NW = _NC * _NS
_B_PER_W = _BATCH // _NW
_CHUNK = 128
_N_CHUNKS = _B_PER_W // _CHUNK
_PASS_CHUNKS = 2

_TCOLS = 2048
_THALF = _TCOLS // 2
_TGRID = -(-_NUM_CLASSES // _TCOLS)

_mesh = plsc.VectorSubcoreMesh(core_axis_name="c", subcore_axis_name="s")


def _tc_transpose_kernel(x_ref, o_ref):
    eye = jnp.eye(_DIM, dtype=jnp.float32)
    zero = jnp.zeros((_DIM, _DIM), dtype=jnp.float32)
    e1 = jnp.concatenate([eye, zero], axis=1)
    e2 = jnp.concatenate([zero, eye], axis=1)
    dn = (((0,), (0,)), ((), ()))
    a = x_ref[:, 0:_THALF]
    b = x_ref[:, _THALF:_TCOLS]
    o_ref[...] = jax.lax.dot_general(
        a, e1, dimension_numbers=dn,
        precision=jax.lax.Precision.HIGHEST,
        preferred_element_type=jnp.float32,
    ) + jax.lax.dot_general(
        b, e2, dimension_numbers=dn,
        precision=jax.lax.Precision.HIGHEST,
        preferred_element_type=jnp.float32,
    )


def _tc_transpose(table_t):
    return pl.pallas_call(
        _tc_transpose_kernel,
        grid=(_TGRID,),
        in_specs=[pl.BlockSpec((_DIM, _TCOLS), lambda j: (0, j))],
        out_specs=pl.BlockSpec((_TCOLS // 2, 2 * _DIM), lambda j: (j, 0)),
        out_shape=jax.ShapeDtypeStruct(
            (_TGRID * _TCOLS // 2, 2 * _DIM), jnp.float32
        ),
    )(table_t)


@functools.partial(
    pl.kernel,
    mesh=_mesh,
    out_type=jax.ShapeDtypeStruct((_BATCH, _DIM), jnp.float32),
    scratch_types=[
        pltpu.VMEM((_B_PER_W,), jnp.int32),
        pltpu.VMEM((_B_PER_W,), jnp.int32),
        pltpu.VMEM((_PASS_CHUNKS * _CHUNK, 2 * _DIM), jnp.float32),
        pltpu.VMEM((_B_PER_W, _DIM), jnp.float32),
        pltpu.SemaphoreType.DMA,
    ],
)
def _embed(y_hbm, table2_hbm, out_hbm, idx_v, q_v, gbuf, rows_v, sem):
    wid = lax.axis_index("s") * _NC + lax.axis_index("c")
    base = wid * _B_PER_W
    pltpu.sync_copy(y_hbm.at[pl.ds(base, _B_PER_W)], idx_v)

    def q_body(g, _):
        sl = pl.ds(g * 16, 16)
        vec = idx_v[sl]
        blk = jax.lax.shift_right_logical(vec, 11)
        q_v[sl] = blk * _THALF + jax.lax.bitwise_and(vec, _THALF - 1)
        return _

    lax.fori_loop(0, _B_PER_W // 16, q_body, None)

    for p in range(_N_CHUNKS // _PASS_CHUNKS):
        pbase = p * _PASS_CHUNKS * _CHUNK
        copies = []
        for j in range(_PASS_CHUNKS):
            copies.append(
                pltpu.async_copy(
                    table2_hbm.at[q_v.at[pl.ds(pbase + j * _CHUNK, _CHUNK)]],
                    gbuf.at[pl.ds(j * _CHUNK, _CHUNK)],
                    sem,
                )
            )
        for c in copies:
            c.wait()

        def sel_body(g, _):
            vec = idx_v[pl.ds(pbase + g * 16, 16)]
            for k in range(16):
                j = g * 16 + k
                h = jax.lax.bitwise_and(
                    jax.lax.shift_right_logical(vec[k], 10), 1
                ) * _DIM
                for q in range(4):
                    rows_v[pbase + j, pl.ds(q * 16, 16)] = gbuf[
                        j, pl.ds(h + q * 16, 16)
                    ]
            return _

        lax.fori_loop(0, _PASS_CHUNKS * _CHUNK // 16, sel_body, None)

    pltpu.sync_copy(rows_v, out_hbm.at[pl.ds(base, _B_PER_W)])


def kernel(y, emb_weight):
    assert y.shape == (_BATCH,) and emb_weight.shape == (_NUM_CLASSES, _DIM)
    table2 = _tc_transpose(emb_weight.T)
    return _embed(y.astype(jnp.int32), table2)


# speedup vs baseline: 2.3010x; 1.2132x over previous
import functools
import jax
import jax.numpy as jnp
from jax import lax
from jax.experimental import pallas as pl
from jax.experimental.pallas import tpu as pltpu
from jax.experimental.pallas import tpu_sc as plsc

_NUM_CLASSES = 1000000
_DIM = 64
_BATCH = 16384

_info = plsc.get_sparse_core_info()
_NC, _NS = _info.num_cores, _info.num_subcores
_NW = _NC * _NS
_B_PER_W = _BATCH // _NW
_CHUNK = 128
_N_CHUNKS = _B_PER_W // _CHUNK
_PASS_CHUNKS = 2

_TCOLS = 8192
_THALF = _TCOLS // 2
_TGRID = -(-_NUM_CLASSES // _TCOLS)

_mesh = plsc.VectorSubcoreMesh(core_axis_name="c", subcore_axis_name="s")


def _tc_transpose_kernel(x_ref, o_ref):
    eye = jnp.eye(_DIM, dtype=jnp.float32)
    zero = jnp.zeros((_DIM, _DIM), dtype=jnp.float32)
    e1 = jnp.concatenate([eye, zero], axis=1)
    e2 = jnp.concatenate([zero, eye], axis=1)
    dn = (((0,), (0,)), ((), ()))
    a = x_ref[:, 0:_THALF]
    b = x_ref[:, _THALF:_TCOLS]
    o_ref[...] = jax.lax.dot_general(
        a, e1, dimension_numbers=dn,
        precision=jax.lax.Precision.HIGHEST,
        preferred_element_type=jnp.float32,
    ) + jax.lax.dot_general(
        b, e2, dimension_numbers=dn,
        precision=jax.lax.Precision.HIGHEST,
        preferred_element_type=jnp.float32,
    )


def _tc_transpose(table_t):
    return pl.pallas_call(
        _tc_transpose_kernel,
        grid=(_TGRID,),
        in_specs=[pl.BlockSpec((_DIM, _TCOLS), lambda j: (0, j))],
        out_specs=pl.BlockSpec((_TCOLS // 2, 2 * _DIM), lambda j: (j, 0)),
        out_shape=jax.ShapeDtypeStruct(
            (_TGRID * _TCOLS // 2, 2 * _DIM), jnp.float32
        ),
    )(table_t)


@functools.partial(
    pl.kernel,
    mesh=_mesh,
    out_type=jax.ShapeDtypeStruct((_BATCH, _DIM), jnp.float32),
    scratch_types=[
        pltpu.VMEM((_B_PER_W,), jnp.int32),
        pltpu.VMEM((_B_PER_W,), jnp.int32),
        pltpu.VMEM((_PASS_CHUNKS * _CHUNK, 2 * _DIM), jnp.float32),
        pltpu.VMEM((_B_PER_W, _DIM), jnp.float32),
        pltpu.SemaphoreType.DMA,
    ],
)
def _embed(y_hbm, table2_hbm, out_hbm, idx_v, q_v, gbuf, rows_v, sem):
    wid = lax.axis_index("s") * _NC + lax.axis_index("c")
    base = wid * _B_PER_W
    pltpu.sync_copy(y_hbm.at[pl.ds(base, _B_PER_W)], idx_v)

    def q_body(g, _):
        sl = pl.ds(g * 16, 16)
        vec = idx_v[sl]
        blk = jax.lax.shift_right_logical(vec, 13)
        q_v[sl] = blk * _THALF + jax.lax.bitwise_and(vec, _THALF - 1)
        return _

    lax.fori_loop(0, _B_PER_W // 16, q_body, None)

    for p in range(_N_CHUNKS // _PASS_CHUNKS):
        pbase = p * _PASS_CHUNKS * _CHUNK
        copies = []
        for j in range(_PASS_CHUNKS):
            copies.append(
                pltpu.async_copy(
                    table2_hbm.at[q_v.at[pl.ds(pbase + j * _CHUNK, _CHUNK)]],
                    gbuf.at[pl.ds(j * _CHUNK, _CHUNK)],
                    sem,
                )
            )
        for c in copies:
            c.wait()

        def sel_body(g, _):
            vec = idx_v[pl.ds(pbase + g * 16, 16)]
            for k in range(16):
                j = g * 16 + k
                h = jax.lax.bitwise_and(
                    jax.lax.shift_right_logical(vec[k], 12), 1
                ) * _DIM
                for q in range(4):
                    rows_v[pbase + j, pl.ds(q * 16, 16)] = gbuf[
                        j, pl.ds(h + q * 16, 16)
                    ]
            return _

        lax.fori_loop(0, _PASS_CHUNKS * _CHUNK // 16, sel_body, None)

    pltpu.sync_copy(rows_v, out_hbm.at[pl.ds(base, _B_PER_W)])


def kernel(y, emb_weight):
    assert y.shape == (_BATCH,) and emb_weight.shape == (_NUM_CLASSES, _DIM)
    table2 = _tc_transpose(emb_weight.T)
    return _embed(y.astype(jnp.int32), table2)


# speedup vs baseline: 3.3955x; 1.4757x over previous
import functools
import jax
import jax.numpy as jnp
from jax import lax
from jax.experimental import pallas as pl
from jax.experimental.pallas import tpu as pltpu
from jax.experimental.pallas import tpu_sc as plsc

_NUM_CLASSES = 1000000
_DIM = 64
_BATCH = 16384

_info = plsc.get_sparse_core_info()
_NC, _NS = _info.num_cores, _info.num_subcores
_NW = _NC * _NS
_B_PER_W = _BATCH // _NW

_mesh = plsc.VectorSubcoreMesh(core_axis_name="c", subcore_axis_name="s")


@functools.partial(
    pl.kernel,
    mesh=_mesh,
    out_type=jax.ShapeDtypeStruct((_BATCH, _DIM), jnp.float32),
    scratch_types=[
        pltpu.VMEM((_B_PER_W,), jnp.int32),
        pltpu.VMEM((_B_PER_W, _DIM), jnp.float32),
        pltpu.SemaphoreType.DMA,
        pltpu.SemaphoreType.DMA,
    ],
)
def _embed(y_hbm, table_hbm, out_hbm, idx_v, rows_v, gsem, osem):
    wid = lax.axis_index("s") * _NC + lax.axis_index("c")
    base = wid * _B_PER_W
    pltpu.make_async_copy(y_hbm.at[pl.ds(base, _B_PER_W)], idx_v, gsem).start()
    pltpu.make_async_copy(y_hbm.at[pl.ds(base, _B_PER_W)], idx_v, gsem).wait()

    def body(g, _):
        vec = idx_v[pl.ds(g * 16, 16)]
        for k in range(16):
            i = vec[k]
            pltpu.make_async_copy(
                table_hbm.at[i], rows_v.at[g * 16 + k], gsem
            ).start()
        return _

    lax.fori_loop(0, _B_PER_W // 16, body, None)

    pltpu.make_async_copy(
        table_hbm.at[pl.ds(0, _B_PER_W)], rows_v, gsem
    ).wait()

    pltpu.make_async_copy(rows_v, out_hbm.at[pl.ds(base, _B_PER_W)], osem).start()
    pltpu.make_async_copy(rows_v, out_hbm.at[pl.ds(base, _B_PER_W)], osem).wait()


def kernel(y, emb_weight):
    assert y.shape == (_BATCH,) and emb_weight.shape == (_NUM_CLASSES, _DIM)
    return _embed(y.astype(jnp.int32), emb_weight)
